# baseline (speedup 1.0000x reference)
import functools

import jax
import jax.numpy as jnp
from jax import lax
from jax.experimental import pallas as pl
from jax.experimental.pallas import tpu as pltpu
from jax.experimental.pallas import tpu_sc as plsc

B_, N_, D_ = 4, 50000, 128
K_ = 40000
NP_ = 50176
NBLK = 28
BN = NP_ // NBLK

NT = 16
CH_T = NP_ // NT
NVR = CH_T // 16
SROW = 7
NSR = NVR // SROW
GCH = 128
NGF = K_ // GCH
GTAIL = K_ - NGF * GCH


def _gate_keys_body(s_ref, x_ref, y_ref, k_ref):
    j = pl.program_id(0)
    s = s_ref[...]
    gate = 1.0 / (1.0 + jnp.exp(-s))
    y_ref[...] = x_ref[...] * gate[..., None]
    ks = lax.bitcast_convert_type(s, jnp.int32)
    sign = lax.shift_right_arithmetic(ks, 31)
    ka = lax.bitwise_xor(ks, lax.bitwise_or(sign, jnp.int32(-2147483648)))
    kd = lax.bitwise_not(ka)
    pos = j * BN + lax.broadcasted_iota(jnp.int32, (B_, BN), 1)
    k_ref[...] = jnp.where(pos < N_, kd, jnp.int32(-1))


def _gate_and_keys(scores_pad, x):
    return pl.pallas_call(
        _gate_keys_body,
        grid=(NBLK,),
        in_specs=[
            pl.BlockSpec((B_, BN), lambda j: (0, j)),
            pl.BlockSpec((B_, BN, D_), lambda j: (0, j, 0)),
        ],
        out_specs=[
            pl.BlockSpec((B_, BN, D_), lambda j: (0, j, 0)),
            pl.BlockSpec((B_, BN), lambda j: (0, j)),
        ],
        out_shape=[
            jax.ShapeDtypeStruct((B_, NP_, D_), jnp.float32),
            jax.ShapeDtypeStruct((B_, NP_), jnp.int32),
        ],
    )(scores_pad, x)


def _lane():
    return lax.iota(jnp.int32, 16)


def _seg_info(sd, sd_s):
    lane = _lane()
    sd_s[...] = sd
    prev = plsc.load_gather(sd_s, [jnp.maximum(lane - 1, 0)])
    is_start = jnp.logical_or(lane == 0, prev != sd)
    seg_start = plsc.cummax(jnp.where(is_start, lane, 0))
    rank = lane - seg_start
    nxt = plsc.load_gather(sd_s, [jnp.minimum(lane + 1, 15)])
    is_end = jnp.logical_or(lane == 15, nxt != sd)
    return rank, is_end


def _make_sort_gather():
    mesh = plsc.VectorSubcoreMesh(core_axis_name="c", subcore_axis_name="s")

    @functools.partial(
        pl.kernel,
        out_type=jax.ShapeDtypeStruct((B_ * K_, D_), jnp.float32),
        mesh=mesh,
        compiler_params=pltpu.CompilerParams(needs_layout_passes=False),
        scratch_types=[
            pltpu.VMEM_SHARED((NP_,), jnp.int32),
            pltpu.VMEM_SHARED((NP_,), jnp.int32),
            pltpu.VMEM_SHARED((NP_,), jnp.int32),
            pltpu.VMEM_SHARED((NP_,), jnp.int32),
            pltpu.VMEM_SHARED((NT, 256), jnp.int32),
            pltpu.VMEM((CH_T,), jnp.int32),
            pltpu.VMEM((CH_T,), jnp.int32),
            pltpu.VMEM((256,), jnp.int32),
            pltpu.VMEM((16 * 256,), jnp.int32),
            pltpu.VMEM((NT, 256), jnp.int32),
            pltpu.VMEM((16,), jnp.int32),
            pltpu.VMEM((CH_T,), jnp.int32),
            pltpu.VMEM((CH_T,), jnp.int32),
            pltpu.VMEM((NSR, SROW * 16), jnp.int32),
            pltpu.VMEM((GCH,), jnp.int32),
            pltpu.VMEM((GCH, D_), jnp.float32),
            pltpu.VMEM((GCH,), jnp.int32),
            pltpu.VMEM((GCH, D_), jnp.float32),
            pltpu.SemaphoreType.DMA,
            pltpu.SemaphoreType.DMA,
            pltpu.SemaphoreType.DMA,
            pltpu.SemaphoreType.DMA,
            pltpu.SemaphoreType.DMA,
        ],
    )
    def sort_gather(keys_hbm, y_hbm, out_hbm,
                    ak, ai, bk, bi, hist_all,
                    tk, ti, hist, hist16, hga, sd_s,
                    stk, sti, stp, gidx, grows, gidx2, grows2,
                    sem, gsem, gsem2, wsem, wsem2):
        c = lax.axis_index("c")
        t = lax.axis_index("s")
        lane = _lane()

        def zero_hist():
            for i in range(16):
                hist[pl.ds(i * 16, 16)] = jnp.zeros((16,), jnp.int32)

        def load_hist_and_offsets():
            pltpu.sync_copy(hist_all, hga)
            carry = jnp.int32(0)
            for dv in range(16):
                sl = pl.ds(dv * 16, 16)
                tot = jnp.zeros((16,), jnp.int32)
                pre = jnp.zeros((16,), jnp.int32)
                for tt in range(NT):
                    h = hga[tt, sl]
                    tot = tot + h
                    pre = pre + jnp.where(jnp.int32(tt) < t, h, 0)
                incl = plsc.cumsum(tot)
                excl = incl - tot
                hist[sl] = excl + carry + pre
                carry = carry + jnp.sum(tot)

        def radix_pass(r, p, src_k, src_i, dst_k, dst_i, first):
            shift = jnp.uint32(8 * p)
            base = t * CH_T
            if first:
                pltpu.sync_copy(keys_hbm.at[pl.ds(r * NP_ + base, CH_T)], tk)
            else:
                cp1 = pltpu.async_copy(src_k.at[pl.ds(base, CH_T)], tk, sem)
                cp2 = pltpu.async_copy(src_i.at[pl.ds(base, CH_T)], ti, sem)
                cp1.wait()
                cp2.wait()
            def zbody(i, _):
                hist16[pl.ds(i * 16, 16)] = jnp.zeros((16,), jnp.int32)
                return 0

            lax.fori_loop(0, 256, zbody, 0, unroll=False)
            ones = jnp.ones((16,), jnp.int32)

            def hbody(v, _):
                kv = plsc.bitcast(tk[pl.ds(v * 16, 16)], jnp.uint32)
                d = plsc.bitcast((kv >> shift) & jnp.uint32(255), jnp.int32)
                plsc.addupdate_scatter(hist16, [lane * 256 + d], ones)
                return 0

            lax.fori_loop(0, NVR, hbody, 0, unroll=False)

            def rbody(dv, _):
                acc = jnp.zeros((16,), jnp.int32)
                for l in range(16):
                    acc = acc + hist16[pl.ds(l * 256 + dv * 16, 16)]
                hist[pl.ds(dv * 16, 16)] = acc
                return 0

            lax.fori_loop(0, 16, rbody, 0, unroll=False)
            pltpu.sync_copy(hist, hist_all.at[t])
            plsc.subcore_barrier()
            load_hist_and_offsets()
            def pbody(j, _):
                for cc in range(SROW):
                    v = j * SROW + cc
                    kv = tk[pl.ds(v * 16, 16)]
                    if first:
                        iv = (r * NP_ + base + v * 16) + lane
                    else:
                        iv = ti[pl.ds(v * 16, 16)]
                    kvu = plsc.bitcast(kv, jnp.uint32)
                    d = plsc.bitcast((kvu >> shift) & jnp.uint32(255),
                                     jnp.int32)
                    ck = d * 16 + lane
                    sck, skv = plsc.sort_key_val(ck, kv)
                    _s2, siv = plsc.sort_key_val(ck, iv)
                    sd = sck >> 4
                    rank, is_end = _seg_info(sd, sd_s)
                    cur = plsc.load_gather(hist, [sd])
                    pos = cur + rank
                    plsc.addupdate_scatter(hist, [sd], rank + 1, mask=is_end)
                    stk[pl.ds(j * SROW * 16 + cc * 16, 16)] = skv
                    sti[pl.ds(j * SROW * 16 + cc * 16, 16)] = siv
                    stp[j, pl.ds(cc * 16, 16)] = pos
                pltpu.async_copy(stk.at[pl.ds(j * SROW * 16, SROW * 16)],
                                 dst_k.at[stp.at[j]], sem)
                pltpu.async_copy(sti.at[pl.ds(j * SROW * 16, SROW * 16)],
                                 dst_i.at[stp.at[j]], sem)
                return 0

            lax.fori_loop(0, NSR, pbody, 0, unroll=False)
            pltpu.make_async_copy(keys_hbm.at[pl.ds(0, CH_T)], stk, sem).wait()
            pltpu.make_async_copy(keys_hbm.at[pl.ds(0, CH_T)], sti, sem).wait()
            plsc.subcore_barrier()

        def sort_row(r):
            radix_pass(r, 0, None, None, ak, ai, first=True)
            radix_pass(r, 1, ak, ai, bk, bi, first=False)
            radix_pass(r, 2, bk, bi, ak, ai, first=False)
            radix_pass(r, 3, ak, ai, bk, bi, first=False)

            NG_IT = (NGF + NT - 1) // NT

            def fire(i, buf_idx, buf_rows, bsem, wsem_b):
                g = i * NT + t

                @pl.when(g < NGF)
                def _():
                    @pl.when(i >= 2)
                    def _():
                        pltpu.make_async_copy(
                            y_hbm.at[buf_idx_dummy], buf_rows, wsem_b).wait()
                    pltpu.sync_copy(bi.at[pl.ds(g * GCH, GCH)], buf_idx)
                    pltpu.async_copy(y_hbm.at[buf_idx], buf_rows, bsem)

            def retire(i, buf_rows, bsem, wsem_b):
                g = i * NT + t

                @pl.when(g < NGF)
                def _():
                    pltpu.make_async_copy(y_hbm.at[buf_idx_dummy], buf_rows,
                                          bsem).wait()
                    pltpu.async_copy(
                        buf_rows, out_hbm.at[pl.ds(r * K_ + g * GCH, GCH)],
                        wsem_b)

            buf_idx_dummy = gidx
            fire(0, gidx, grows, gsem, wsem)

            def gbody(i, _):
                @pl.when(i % 2 == 0)
                def _():
                    fire(i + 1, gidx2, grows2, gsem2, wsem2)
                    retire(i, grows, gsem, wsem)

                @pl.when(i % 2 == 1)
                def _():
                    fire(i + 1, gidx, grows, gsem, wsem)
                    retire(i, grows2, gsem2, wsem2)
                return 0

            lax.fori_loop(0, NG_IT, gbody, 0, unroll=False)
            pltpu.make_async_copy(y_hbm.at[buf_idx_dummy], grows, wsem).wait()
            pltpu.make_async_copy(y_hbm.at[buf_idx_dummy], grows2, wsem2).wait()

            @pl.when(t == NT - 1)
            def _():
                pltpu.sync_copy(bi.at[pl.ds(NGF * GCH, GTAIL)],
                                gidx.at[pl.ds(0, GTAIL)])
                pltpu.async_copy(y_hbm.at[gidx.at[pl.ds(0, GTAIL)]],
                                 grows.at[pl.ds(0, GTAIL)], sem).wait()
                pltpu.sync_copy(grows.at[pl.ds(0, GTAIL)],
                                out_hbm.at[pl.ds(r * K_ + NGF * GCH, GTAIL)])
            plsc.subcore_barrier()

        sort_row(2 * c)
        sort_row(2 * c + 1)

    return sort_gather


_sort_gather = _make_sort_gather()


def kernel(x, W, b):
    scores = (x @ W + b)[..., 0]
    scores_pad = jnp.pad(scores, ((0, 0), (0, NP_ - N_)))
    y, keys = _gate_and_keys(scores_pad, x)
    y2d = y.reshape(B_ * NP_, D_)
    out = _sort_gather(keys.reshape(B_ * NP_), y2d)
    return out.reshape(B_, K_, D_)


# speedup vs baseline: 1.0022x; 1.0022x over previous
import functools

import jax
import jax.numpy as jnp
from jax import lax
from jax.experimental import pallas as pl
from jax.experimental.pallas import tpu as pltpu
from jax.experimental.pallas import tpu_sc as plsc

B_, N_, D_ = 4, 50000, 128
K_ = 40000
NP_ = 50176
NBLK = 28
BN = NP_ // NBLK

NT = 16
CH_T = NP_ // NT
NVR = CH_T // 16
SROW = 7
NSR = NVR // SROW
GCH = 128
NGF = K_ // GCH
GTAIL = K_ - NGF * GCH


def _gate_keys_body(s_ref, x_ref, y_ref, k_ref):
    j = pl.program_id(0)
    s = s_ref[...]
    gate = 1.0 / (1.0 + jnp.exp(-s))
    y_ref[...] = x_ref[...] * gate[..., None]
    ks = lax.bitcast_convert_type(s, jnp.int32)
    sign = lax.shift_right_arithmetic(ks, 31)
    ka = lax.bitwise_xor(ks, lax.bitwise_or(sign, jnp.int32(-2147483648)))
    kd = lax.bitwise_not(ka)
    pos = j * BN + lax.broadcasted_iota(jnp.int32, (B_, BN), 1)
    k_ref[...] = jnp.where(pos < N_, kd, jnp.int32(-1))


def _gate_and_keys(scores_pad, x):
    return pl.pallas_call(
        _gate_keys_body,
        grid=(NBLK,),
        in_specs=[
            pl.BlockSpec((B_, BN), lambda j: (0, j)),
            pl.BlockSpec((B_, BN, D_), lambda j: (0, j, 0)),
        ],
        out_specs=[
            pl.BlockSpec((B_, BN, D_), lambda j: (0, j, 0)),
            pl.BlockSpec((B_, BN), lambda j: (0, j)),
        ],
        out_shape=[
            jax.ShapeDtypeStruct((B_, NP_, D_), jnp.float32),
            jax.ShapeDtypeStruct((B_, NP_), jnp.int32),
        ],
    )(scores_pad, x)


def _lane():
    return lax.iota(jnp.int32, 16)


def _seg_info(sd, sd_s):
    lane = _lane()
    sd_s[...] = sd
    prev = plsc.load_gather(sd_s, [jnp.maximum(lane - 1, 0)])
    is_start = jnp.logical_or(lane == 0, prev != sd)
    seg_start = plsc.cummax(jnp.where(is_start, lane, 0))
    rank = lane - seg_start
    nxt = plsc.load_gather(sd_s, [jnp.minimum(lane + 1, 15)])
    is_end = jnp.logical_or(lane == 15, nxt != sd)
    return rank, is_end


def _make_sort_gather():
    mesh = plsc.VectorSubcoreMesh(core_axis_name="c", subcore_axis_name="s")

    @functools.partial(
        pl.kernel,
        out_type=jax.ShapeDtypeStruct((B_ * K_, D_), jnp.float32),
        mesh=mesh,
        compiler_params=pltpu.CompilerParams(needs_layout_passes=False),
        scratch_types=[
            pltpu.VMEM_SHARED((NP_,), jnp.int32),
            pltpu.VMEM_SHARED((NP_,), jnp.int32),
            pltpu.VMEM_SHARED((NP_,), jnp.int32),
            pltpu.VMEM_SHARED((NP_,), jnp.int32),
            pltpu.VMEM_SHARED((NT, 256), jnp.int32),
            pltpu.VMEM((CH_T,), jnp.int32),
            pltpu.VMEM((CH_T,), jnp.int32),
            pltpu.VMEM((256,), jnp.int32),
            pltpu.VMEM((16 * 256,), jnp.int32),
            pltpu.VMEM((NT, 256), jnp.int32),
            pltpu.VMEM((16,), jnp.int32),
            pltpu.VMEM((CH_T,), jnp.int32),
            pltpu.VMEM((CH_T,), jnp.int32),
            pltpu.VMEM((NSR, SROW * 16), jnp.int32),
            pltpu.VMEM((GCH,), jnp.int32),
            pltpu.VMEM((GCH, D_), jnp.float32),
            pltpu.VMEM((GCH,), jnp.int32),
            pltpu.VMEM((GCH, D_), jnp.float32),
            pltpu.SemaphoreType.DMA,
            pltpu.SemaphoreType.DMA,
            pltpu.SemaphoreType.DMA,
            pltpu.SemaphoreType.DMA,
            pltpu.SemaphoreType.DMA,
        ],
    )
    def sort_gather(keys_hbm, y_hbm, out_hbm,
                    ak, ai, bk, bi, hist_all,
                    tk, ti, hist, hist16, hga, sd_s,
                    stk, sti, stp, gidx, grows, gidx2, grows2,
                    sem, gsem, gsem2, wsem, wsem2):
        c = lax.axis_index("c")
        t = lax.axis_index("s")
        lane = _lane()

        def load_hist_and_offsets():
            pltpu.sync_copy(hist_all, hga)
            carry = jnp.int32(0)
            for dv in range(16):
                sl = pl.ds(dv * 16, 16)
                tot = jnp.zeros((16,), jnp.int32)
                pre = jnp.zeros((16,), jnp.int32)
                for tt in range(NT):
                    h = hga[tt, sl]
                    tot = tot + h
                    pre = pre + jnp.where(jnp.int32(tt) < t, h, 0)
                incl = plsc.cumsum(tot)
                excl = incl - tot
                hist[sl] = excl + carry + pre
                carry = carry + jnp.sum(tot)

        def radix_pass(r, p, src_k, src_i, dst_k, dst_i, first):
            shift = jnp.uint32(8 * p)
            base = t * CH_T
            if first:
                pltpu.sync_copy(keys_hbm.at[pl.ds(r * NP_ + base, CH_T)], tk)
            else:
                cp1 = pltpu.async_copy(src_k.at[pl.ds(base, CH_T)], tk, sem)
                cp2 = pltpu.async_copy(src_i.at[pl.ds(base, CH_T)], ti, sem)
                cp1.wait()
                cp2.wait()
            def zbody(i, _):
                hist16[pl.ds(i * 16, 16)] = jnp.zeros((16,), jnp.int32)
                return 0

            lax.fori_loop(0, 256, zbody, 0, unroll=False)
            ones = jnp.ones((16,), jnp.int32)

            def hbody(v, _):
                kv = plsc.bitcast(tk[pl.ds(v * 16, 16)], jnp.uint32)
                d = plsc.bitcast((kv >> shift) & jnp.uint32(255), jnp.int32)
                plsc.addupdate_scatter(hist16, [lane * 256 + d], ones)
                return 0

            lax.fori_loop(0, NVR, hbody, 0, unroll=False)

            def rbody(dv, _):
                acc = jnp.zeros((16,), jnp.int32)
                for l in range(16):
                    acc = acc + hist16[pl.ds(l * 256 + dv * 16, 16)]
                hist[pl.ds(dv * 16, 16)] = acc
                return 0

            lax.fori_loop(0, 16, rbody, 0, unroll=False)
            pltpu.sync_copy(hist, hist_all.at[t])
            plsc.subcore_barrier()
            load_hist_and_offsets()
            def pbody(j, _):
                for cc in range(SROW):
                    v = j * SROW + cc
                    kv = tk[pl.ds(v * 16, 16)]
                    if first:
                        iv = (r * NP_ + base + v * 16) + lane
                    else:
                        iv = ti[pl.ds(v * 16, 16)]
                    kvu = plsc.bitcast(kv, jnp.uint32)
                    d = plsc.bitcast((kvu >> shift) & jnp.uint32(255),
                                     jnp.int32)
                    ck = d * 16 + lane
                    sck, skv = plsc.sort_key_val(ck, kv)
                    _s2, siv = plsc.sort_key_val(ck, iv)
                    sd = sck >> 4
                    rank, is_end = _seg_info(sd, sd_s)
                    cur = plsc.load_gather(hist, [sd])
                    pos = cur + rank
                    plsc.addupdate_scatter(hist, [sd], rank + 1, mask=is_end)
                    stk[pl.ds(j * SROW * 16 + cc * 16, 16)] = skv
                    sti[pl.ds(j * SROW * 16 + cc * 16, 16)] = siv
                    stp[j, pl.ds(cc * 16, 16)] = pos
                pltpu.async_copy(stk.at[pl.ds(j * SROW * 16, SROW * 16)],
                                 dst_k.at[stp.at[j]], sem)
                pltpu.async_copy(sti.at[pl.ds(j * SROW * 16, SROW * 16)],
                                 dst_i.at[stp.at[j]], sem)
                return 0

            lax.fori_loop(0, NSR, pbody, 0, unroll=False)
            pltpu.make_async_copy(keys_hbm.at[pl.ds(0, CH_T)], stk, sem).wait()
            pltpu.make_async_copy(keys_hbm.at[pl.ds(0, CH_T)], sti, sem).wait()
            plsc.subcore_barrier()

        def sort_row(r):
            radix_pass(r, 0, None, None, ak, ai, first=True)
            radix_pass(r, 1, ak, ai, bk, bi, first=False)
            radix_pass(r, 2, bk, bi, ak, ai, first=False)
            radix_pass(r, 3, ak, ai, bk, bi, first=False)

            NG_IT = (NGF + NT - 1) // NT

            def fire(i, buf_idx, buf_rows, bsem, wsem_b):
                g = i * NT + t

                @pl.when(g < NGF)
                def _():
                    @pl.when(i >= 2)
                    def _():
                        pltpu.make_async_copy(
                            y_hbm.at[buf_idx_dummy], buf_rows, wsem_b).wait()
                    pltpu.sync_copy(bi.at[pl.ds(g * GCH, GCH)], buf_idx)
                    pltpu.async_copy(y_hbm.at[buf_idx], buf_rows, bsem)

            def retire(i, buf_rows, bsem, wsem_b):
                g = i * NT + t

                @pl.when(g < NGF)
                def _():
                    pltpu.make_async_copy(y_hbm.at[buf_idx_dummy], buf_rows,
                                          bsem).wait()
                    pltpu.async_copy(
                        buf_rows, out_hbm.at[pl.ds(r * K_ + g * GCH, GCH)],
                        wsem_b)

            buf_idx_dummy = gidx
            fire(0, gidx, grows, gsem, wsem)

            def gbody(i, _):
                @pl.when(i % 2 == 0)
                def _():
                    fire(i + 1, gidx2, grows2, gsem2, wsem2)
                    retire(i, grows, gsem, wsem)

                @pl.when(i % 2 == 1)
                def _():
                    fire(i + 1, gidx, grows, gsem, wsem)
                    retire(i, grows2, gsem2, wsem2)
                return 0

            lax.fori_loop(0, NG_IT, gbody, 0, unroll=False)
            pltpu.make_async_copy(y_hbm.at[buf_idx_dummy], grows, wsem).wait()
            pltpu.make_async_copy(y_hbm.at[buf_idx_dummy], grows2, wsem2).wait()

            @pl.when(t == NT - 1)
            def _():
                pltpu.sync_copy(bi.at[pl.ds(NGF * GCH, GTAIL)],
                                gidx.at[pl.ds(0, GTAIL)])
                pltpu.async_copy(y_hbm.at[gidx.at[pl.ds(0, GTAIL)]],
                                 grows.at[pl.ds(0, GTAIL)], sem).wait()
                pltpu.sync_copy(grows.at[pl.ds(0, GTAIL)],
                                out_hbm.at[pl.ds(r * K_ + NGF * GCH, GTAIL)])
            plsc.subcore_barrier()

        sort_row(2 * c)
        sort_row(2 * c + 1)

    return sort_gather


_sort_gather = _make_sort_gather()


def kernel(x, W, b):
    scores = (x @ W + b)[..., 0]
    scores_pad = jnp.pad(scores, ((0, 0), (0, NP_ - N_)))
    y, keys = _gate_and_keys(scores_pad, x)
    y2d = y.reshape(B_ * NP_, D_)
    out = _sort_gather(keys.reshape(B_ * NP_), y2d)
    return out.reshape(B_, K_, D_)
